# baseline (speedup 1.0000x reference)
import functools

import jax
import jax.numpy as jnp
from jax import lax
from jax.experimental import pallas as pl
from jax.experimental.pallas import tpu as pltpu
from jax.experimental.pallas import tpu_sc as plsc


def _sc_gather(table, idx):
    V, D = table.shape
    (B,) = idx.shape
    info = plsc.get_sparse_core_info()
    NC, NS = info.num_cores, info.num_subcores
    NW = NC * NS
    assert B % NW == 0
    b_per_w = B // NW
    mesh = plsc.VectorSubcoreMesh(core_axis_name="c", subcore_axis_name="s")

    @functools.partial(
        pl.kernel,
        mesh=mesh,
        out_type=jax.ShapeDtypeStruct((B, D), jnp.float32),
        scratch_types=[
            pltpu.VMEM((b_per_w,), jnp.int32),
            pltpu.VMEM((b_per_w, D), jnp.float32),
            pltpu.SemaphoreType.DMA,
        ],
    )
    def gather_kernel(table_hbm, idx_hbm, out_hbm, idx_v, rows_v, sem):
        wid = lax.axis_index("s") * NC + lax.axis_index("c")
        base = wid * b_per_w
        pltpu.sync_copy(idx_hbm.at[pl.ds(base, b_per_w)], idx_v)
        pltpu.async_copy(table_hbm.at[idx_v], rows_v, sem).wait()
        pltpu.sync_copy(rows_v, out_hbm.at[pl.ds(base, b_per_w)])

    return gather_kernel(table, idx)


def _mlp_body(x_ref, s_ref, b_ref, w1_ref, b1_ref, w2_ref, b2_ref,
              o_ref, xln_ref):
    k = pl.program_id(0)

    @pl.when(k == 0)
    def _():
        x = x_ref[...]
        mean = jnp.mean(x, axis=1, keepdims=True)
        var = jnp.mean((x - mean) ** 2, axis=1, keepdims=True)
        xln = (x - mean) * lax.rsqrt(var + 1e-5) * s_ref[...] + b_ref[...]
        xln_ref[...] = xln
        o_ref[...] = jnp.broadcast_to(b2_ref[...], o_ref.shape)

    h = jnp.dot(xln_ref[...], w1_ref[...],
                preferred_element_type=jnp.float32,
                precision=lax.Precision.DEFAULT)
    h = jax.nn.gelu(h + b1_ref[...])
    o_ref[...] += jnp.dot(h, w2_ref[...],
                          preferred_element_type=jnp.float32,
                          precision=lax.Precision.DEFAULT)


def _tc_mlp(x, ln_scale, ln_bias, W1, b1, W2, b2):
    N, D = x.shape
    H = W1.shape[1]
    BLK = 256
    grid = H // BLK
    return pl.pallas_call(
        _mlp_body,
        grid=(grid,),
        in_specs=[
            pl.BlockSpec((N, D), lambda k: (0, 0)),
            pl.BlockSpec((1, D), lambda k: (0, 0)),
            pl.BlockSpec((1, D), lambda k: (0, 0)),
            pl.BlockSpec((D, BLK), lambda k: (0, k)),
            pl.BlockSpec((1, BLK), lambda k: (0, k)),
            pl.BlockSpec((BLK, D), lambda k: (k, 0)),
            pl.BlockSpec((1, D), lambda k: (0, 0)),
        ],
        out_specs=pl.BlockSpec((N, D), lambda k: (0, 0)),
        out_shape=jax.ShapeDtypeStruct((N, D), jnp.float32),
        scratch_shapes=[pltpu.VMEM((N, D), jnp.float32)],
        compiler_params=pltpu.CompilerParams(
            dimension_semantics=("arbitrary",),
        ),
    )(x, ln_scale.reshape(1, D), ln_bias.reshape(1, D),
      W1, b1.reshape(1, H), W2, b2.reshape(1, D))


def kernel(index, codebook, ln_scale, ln_bias, W1, b1, W2, b2):
    Bb, M = index.shape
    V, D = codebook.shape
    idx_flat = index.reshape(-1).astype(jnp.int32)
    x = _sc_gather(codebook, idx_flat)
    rec = _tc_mlp(x, ln_scale, ln_bias, W1, b1, W2, b2)
    return rec.reshape(Bb, M, D)


# speedup vs baseline: 1.2417x; 1.2417x over previous
import functools

import jax
import jax.numpy as jnp
from jax import lax
from jax.experimental import pallas as pl
from jax.experimental.pallas import tpu as pltpu
from jax.experimental.pallas import tpu_sc as plsc


def _sc_gather(table, idx):
    V, D = table.shape
    (B,) = idx.shape
    info = plsc.get_sparse_core_info()
    NC, NS = info.num_cores, info.num_subcores
    NW = NC * NS
    assert B % NW == 0
    b_per_w = B // NW
    mesh = plsc.VectorSubcoreMesh(core_axis_name="c", subcore_axis_name="s")

    @functools.partial(
        pl.kernel,
        mesh=mesh,
        out_type=jax.ShapeDtypeStruct((B, D), jnp.float32),
        scratch_types=[
            pltpu.VMEM((b_per_w,), jnp.int32),
            pltpu.VMEM((b_per_w, D), jnp.float32),
            pltpu.SemaphoreType.DMA,
        ],
    )
    def gather_kernel(table_hbm, idx_hbm, out_hbm, idx_v, rows_v, sem):
        wid = lax.axis_index("s") * NC + lax.axis_index("c")
        base = wid * b_per_w
        pltpu.sync_copy(idx_hbm.at[pl.ds(base, b_per_w)], idx_v)
        pltpu.async_copy(table_hbm.at[idx_v], rows_v, sem).wait()
        pltpu.sync_copy(rows_v, out_hbm.at[pl.ds(base, b_per_w)])

    return gather_kernel(table, idx)


def _mlp_body(x_ref, s_ref, b_ref, w1_ref, b1_ref, w2_ref, b2_ref,
              o_ref, xln_ref):
    k = pl.program_id(0)

    @pl.when(k == 0)
    def _():
        x = x_ref[...]
        mean = jnp.mean(x, axis=1, keepdims=True)
        var = jnp.mean((x - mean) ** 2, axis=1, keepdims=True)
        xln = (x - mean) * lax.rsqrt(var + 1e-5) * s_ref[...] + b_ref[...]
        xln_ref[...] = xln
        o_ref[...] = jnp.broadcast_to(b2_ref[...], o_ref.shape)

    h = jnp.dot(xln_ref[...], w1_ref[...],
                preferred_element_type=jnp.float32,
                precision=lax.Precision.DEFAULT)
    h = jax.nn.gelu(h + b1_ref[...])
    o_ref[...] += jnp.dot(h, w2_ref[...],
                          preferred_element_type=jnp.float32,
                          precision=lax.Precision.DEFAULT)


def _tc_mlp(x, ln_scale, ln_bias, W1, b1, W2, b2):
    N, D = x.shape
    H = W1.shape[1]
    BLK = 2048
    grid = H // BLK
    return pl.pallas_call(
        _mlp_body,
        grid=(grid,),
        in_specs=[
            pl.BlockSpec((N, D), lambda k: (0, 0)),
            pl.BlockSpec((1, D), lambda k: (0, 0)),
            pl.BlockSpec((1, D), lambda k: (0, 0)),
            pl.BlockSpec((D, BLK), lambda k: (0, k)),
            pl.BlockSpec((1, BLK), lambda k: (0, k)),
            pl.BlockSpec((BLK, D), lambda k: (k, 0)),
            pl.BlockSpec((1, D), lambda k: (0, 0)),
        ],
        out_specs=pl.BlockSpec((N, D), lambda k: (0, 0)),
        out_shape=jax.ShapeDtypeStruct((N, D), jnp.float32),
        scratch_shapes=[pltpu.VMEM((N, D), jnp.float32)],
        compiler_params=pltpu.CompilerParams(
            dimension_semantics=("arbitrary",),
        ),
    )(x, ln_scale.reshape(1, D), ln_bias.reshape(1, D),
      W1, b1.reshape(1, H), W2, b2.reshape(1, D))


def kernel(index, codebook, ln_scale, ln_bias, W1, b1, W2, b2):
    Bb, M = index.shape
    V, D = codebook.shape
    idx_flat = index.reshape(-1).astype(jnp.int32)
    x = _sc_gather(codebook, idx_flat)
    rec = _tc_mlp(x, ln_scale, ln_bias, W1, b1, W2, b2)
    return rec.reshape(Bb, M, D)


# speedup vs baseline: 1.3369x; 1.0767x over previous
import functools

import jax
import jax.numpy as jnp
from jax import lax
from jax.experimental import pallas as pl
from jax.experimental.pallas import tpu as pltpu
from jax.experimental.pallas import tpu_sc as plsc


def _sc_gather(table, idx):
    V, D = table.shape
    (B,) = idx.shape
    info = plsc.get_sparse_core_info()
    NC, NS = info.num_cores, info.num_subcores
    NW = NC * NS
    assert B % NW == 0
    b_per_w = B // NW
    mesh = plsc.VectorSubcoreMesh(core_axis_name="c", subcore_axis_name="s")

    @functools.partial(
        pl.kernel,
        mesh=mesh,
        out_type=jax.ShapeDtypeStruct((B, D), jnp.float32),
        scratch_types=[
            pltpu.VMEM((b_per_w,), jnp.int32),
            pltpu.VMEM((b_per_w, D), jnp.float32),
            pltpu.SemaphoreType.DMA,
        ],
    )
    def gather_kernel(table_hbm, idx_hbm, out_hbm, idx_v, rows_v, sem):
        wid = lax.axis_index("s") * NC + lax.axis_index("c")
        base = wid * b_per_w
        pltpu.sync_copy(idx_hbm.at[pl.ds(base, b_per_w)], idx_v)
        pltpu.async_copy(table_hbm.at[idx_v], rows_v, sem).wait()
        pltpu.sync_copy(rows_v, out_hbm.at[pl.ds(base, b_per_w)])

    return gather_kernel(table, idx)


_MLP_BLK = 1024


def _mlp_body(x_ref, s_ref, b_ref, b1_ref, b2_ref, w1_hbm, w2_hbm,
              o_ref, xln_ref, w1b, w2b, sem1, sem2):
    BLK = _MLP_BLK
    H = w1_hbm.shape[1]
    K = H // BLK

    def w1_copy(k, slot):
        return pltpu.make_async_copy(
            w1_hbm.at[:, pl.ds(k * BLK, BLK)], w1b.at[slot], sem1.at[slot])

    def w2_copy(k, slot):
        return pltpu.make_async_copy(
            w2_hbm.at[pl.ds(k * BLK, BLK), :], w2b.at[slot], sem2.at[slot])

    w1_copy(0, 0).start()
    w2_copy(0, 0).start()

    x = x_ref[...]
    mean = jnp.mean(x, axis=1, keepdims=True)
    var = jnp.mean((x - mean) ** 2, axis=1, keepdims=True)
    xln_ref[...] = (x - mean) * lax.rsqrt(var + 1e-5) * s_ref[...] + b_ref[...]
    o_ref[...] = jnp.broadcast_to(b2_ref[...], o_ref.shape)

    for k in range(K):
        slot = k % 2
        if k + 1 < K:
            w1_copy(k + 1, 1 - slot).start()
            w2_copy(k + 1, 1 - slot).start()
        w1_copy(k, slot).wait()
        w2_copy(k, slot).wait()
        h = jnp.dot(xln_ref[...], w1b[slot],
                    preferred_element_type=jnp.float32)
        h = jax.nn.gelu(h + b1_ref[:, pl.ds(k * BLK, BLK)])
        o_ref[...] += jnp.dot(h, w2b[slot],
                              preferred_element_type=jnp.float32)


def _tc_mlp(x, ln_scale, ln_bias, W1, b1, W2, b2):
    N, D = x.shape
    H = W1.shape[1]
    BLK = _MLP_BLK
    return pl.pallas_call(
        _mlp_body,
        in_specs=[
            pl.BlockSpec(memory_space=pltpu.VMEM),
            pl.BlockSpec(memory_space=pltpu.VMEM),
            pl.BlockSpec(memory_space=pltpu.VMEM),
            pl.BlockSpec(memory_space=pltpu.VMEM),
            pl.BlockSpec(memory_space=pltpu.VMEM),
            pl.BlockSpec(memory_space=pl.ANY),
            pl.BlockSpec(memory_space=pl.ANY),
        ],
        out_specs=pl.BlockSpec(memory_space=pltpu.VMEM),
        out_shape=jax.ShapeDtypeStruct((N, D), jnp.float32),
        scratch_shapes=[
            pltpu.VMEM((N, D), jnp.float32),
            pltpu.VMEM((2, D, BLK), jnp.float32),
            pltpu.VMEM((2, BLK, D), jnp.float32),
            pltpu.SemaphoreType.DMA((2,)),
            pltpu.SemaphoreType.DMA((2,)),
        ],
    )(x, ln_scale.reshape(1, D), ln_bias.reshape(1, D),
      b1.reshape(1, H), b2.reshape(1, D), W1, W2)


def kernel(index, codebook, ln_scale, ln_bias, W1, b1, W2, b2):
    Bb, M = index.shape
    V, D = codebook.shape
    idx_flat = index.reshape(-1).astype(jnp.int32)
    x = _sc_gather(codebook, idx_flat)
    rec = _tc_mlp(x, ln_scale, ln_bias, W1, b1, W2, b2)
    return rec.reshape(Bb, M, D)


# speedup vs baseline: 1.3375x; 1.0005x over previous
import functools

import jax
import jax.numpy as jnp
from jax import lax
from jax.experimental import pallas as pl
from jax.experimental.pallas import tpu as pltpu
from jax.experimental.pallas import tpu_sc as plsc


def _sc_gather(table, idx):
    V, D = table.shape
    (B,) = idx.shape
    info = plsc.get_sparse_core_info()
    NC, NS = info.num_cores, info.num_subcores
    NW = NC * NS
    assert B % NW == 0
    b_per_w = B // NW
    mesh = plsc.VectorSubcoreMesh(core_axis_name="c", subcore_axis_name="s")

    @functools.partial(
        pl.kernel,
        mesh=mesh,
        out_type=jax.ShapeDtypeStruct((B, D), jnp.float32),
        scratch_types=[
            pltpu.VMEM((b_per_w,), jnp.int32),
            pltpu.VMEM((b_per_w, D), jnp.float32),
            pltpu.SemaphoreType.DMA,
        ],
    )
    def gather_kernel(table_hbm, idx_hbm, out_hbm, idx_v, rows_v, sem):
        wid = lax.axis_index("s") * NC + lax.axis_index("c")
        base = wid * b_per_w
        pltpu.sync_copy(idx_hbm.at[pl.ds(base, b_per_w)], idx_v)
        pltpu.async_copy(table_hbm.at[idx_v], rows_v, sem).wait()
        pltpu.sync_copy(rows_v, out_hbm.at[pl.ds(base, b_per_w)])

    return gather_kernel(table, idx)


_MLP_BLK = 1024


def _mlp_body(x_ref, s_ref, b_ref, b1_ref, b2_ref, w1_hbm, w2_hbm,
              o_ref, xln_ref, w1b, w2b, sem1, sem2):
    BLK = _MLP_BLK
    H = w1_hbm.shape[1]
    K = H // BLK

    def w1_copy(k, slot):
        return pltpu.make_async_copy(
            w1_hbm.at[:, pl.ds(k * BLK, BLK)], w1b.at[slot], sem1.at[slot])

    def w2_copy(k, slot):
        return pltpu.make_async_copy(
            w2_hbm.at[pl.ds(k * BLK, BLK), :], w2b.at[slot], sem2.at[slot])

    w1_copy(0, 0).start()
    w2_copy(0, 0).start()
    w1_copy(1, 1).start()
    w2_copy(1, 1).start()

    x = x_ref[...]
    mean = jnp.mean(x, axis=1, keepdims=True)
    var = jnp.mean((x - mean) ** 2, axis=1, keepdims=True)
    xln_ref[...] = (x - mean) * lax.rsqrt(var + 1e-5) * s_ref[...] + b_ref[...]
    o_ref[...] = jnp.broadcast_to(b2_ref[...], o_ref.shape)

    for k in range(K):
        slot = k % 2
        w1_copy(k, slot).wait()
        w2_copy(k, slot).wait()
        h = jnp.dot(xln_ref[...], w1b[slot],
                    preferred_element_type=jnp.float32)
        h = jax.nn.gelu(h + b1_ref[:, pl.ds(k * BLK, BLK)])
        o_ref[...] += jnp.dot(h, w2b[slot],
                              preferred_element_type=jnp.float32)
        if k + 2 < K:
            w1_copy(k + 2, slot).start()
            w2_copy(k + 2, slot).start()


def _tc_mlp(x, ln_scale, ln_bias, W1, b1, W2, b2):
    N, D = x.shape
    H = W1.shape[1]
    BLK = _MLP_BLK
    return pl.pallas_call(
        _mlp_body,
        in_specs=[
            pl.BlockSpec(memory_space=pltpu.VMEM),
            pl.BlockSpec(memory_space=pltpu.VMEM),
            pl.BlockSpec(memory_space=pltpu.VMEM),
            pl.BlockSpec(memory_space=pltpu.VMEM),
            pl.BlockSpec(memory_space=pltpu.VMEM),
            pl.BlockSpec(memory_space=pl.ANY),
            pl.BlockSpec(memory_space=pl.ANY),
        ],
        out_specs=pl.BlockSpec(memory_space=pltpu.VMEM),
        out_shape=jax.ShapeDtypeStruct((N, D), jnp.float32),
        scratch_shapes=[
            pltpu.VMEM((N, D), jnp.float32),
            pltpu.VMEM((2, D, BLK), jnp.float32),
            pltpu.VMEM((2, BLK, D), jnp.float32),
            pltpu.SemaphoreType.DMA((2,)),
            pltpu.SemaphoreType.DMA((2,)),
        ],
    )(x, ln_scale.reshape(1, D), ln_bias.reshape(1, D),
      b1.reshape(1, H), b2.reshape(1, D), W1, W2)


def kernel(index, codebook, ln_scale, ln_bias, W1, b1, W2, b2):
    Bb, M = index.shape
    V, D = codebook.shape
    idx_flat = index.reshape(-1).astype(jnp.int32)
    x = _sc_gather(codebook, idx_flat)
    rec = _tc_mlp(x, ln_scale, ln_bias, W1, b1, W2, b2)
    return rec.reshape(Bb, M, D)
